# baseline (speedup 1.0000x reference)
import functools

import jax
import jax.numpy as jnp
from jax import lax
from jax.experimental import pallas as pl
from jax.experimental.pallas import tpu as pltpu
from jax.experimental.pallas import tpu_sc as plsc

D_EMB = 64
D_PAD = 128
D_MODEL = 1024

_NC = 2
_NS = 16
_NW = _NC * _NS

_CB2 = 512
_GB = 128


_VB = 8192


def _tr_body(t_ref, o_ref):
    x = t_ref[...]
    r = lax.broadcasted_iota(jnp.int32, (D_EMB, D_EMB), 0)
    c = lax.broadcasted_iota(jnp.int32, (D_EMB, D_EMB), 1)
    ident = (r == c).astype(jnp.float32)
    o_ref[:, 0:D_EMB] = lax.dot_general(
        x, ident, (((0,), (0,)), ((), ())), preferred_element_type=jnp.float32
    )


def _tc_transpose(table_t):
    v = table_t.shape[1]
    return pl.pallas_call(
        _tr_body,
        grid=(pl.cdiv(v, _VB),),
        in_specs=[pl.BlockSpec((D_EMB, _VB), lambda i: (0, i))],
        out_specs=pl.BlockSpec((_VB, D_PAD), lambda i: (i, 0)),
        out_shape=jax.ShapeDtypeStruct((v, D_PAD), jnp.float32),
    )(table_t)


def _gather_body(ids_hbm, table_hbm, out_hbm, idx_v, rows_v, sem, n_per_w):
    wid = lax.axis_index("s") * _NC + lax.axis_index("c")
    base = wid * n_per_w
    pltpu.sync_copy(ids_hbm.at[pl.ds(base, n_per_w)], idx_v)

    def body(i, carry):
        tok = i * (2 * _CB2)
        for half in range(2):
            for j in range(_CB2 // _GB):
                pltpu.async_copy(
                    table_hbm.at[idx_v.at[pl.ds(tok + half * _CB2 + j * _GB, _GB)]],
                    rows_v.at[pl.ds((half * _CB2 + j * _GB), _GB)],
                    sem,
                )
        for half in range(2):
            for j in range(_CB2 // _GB):
                pltpu.make_async_copy(
                    table_hbm.at[idx_v.at[pl.ds(tok + half * _CB2 + j * _GB, _GB)]],
                    rows_v.at[pl.ds((half * _CB2 + j * _GB), _GB)],
                    sem,
                ).wait()
        prow = base // 2 + i * _CB2
        pltpu.sync_copy(
            rows_v.at[pl.ds(0, _CB2)],
            out_hbm.at[pl.ds(prow, _CB2), pl.ds(0, D_EMB)],
        )
        pltpu.sync_copy(
            rows_v.at[pl.ds(_CB2, _CB2)],
            out_hbm.at[pl.ds(prow, _CB2), pl.ds(D_EMB, D_EMB)],
        )
        return carry

    lax.fori_loop(0, n_per_w // (2 * _CB2), body, 0)


def _sc_gather(ids2, table2):
    n = ids2.shape[0]
    n_per_w = n // _NW
    mesh = plsc.VectorSubcoreMesh(core_axis_name="c", subcore_axis_name="s")
    k = pl.kernel(
        functools.partial(_gather_body, n_per_w=n_per_w),
        out_type=jax.ShapeDtypeStruct((n // 2, 2 * D_EMB), jnp.float32),
        mesh=mesh,
        scratch_types=[
            pltpu.VMEM((n_per_w,), jnp.int32),
            pltpu.VMEM((2 * _CB2, D_EMB), jnp.float32),
            pltpu.SemaphoreType.DMA,
        ],
        compiler_params=pltpu.CompilerParams(use_tc_tiling_on_sc=False),
    )
    return k(ids2, table2)


_G = 4


def _proj_body(e2_ref, w_ref, o_ref):
    p = e2_ref[...]
    w = w_ref[...]
    dn = (((1,), (1,)), ((), ()))
    lo = lax.dot_general(p[:, 0:D_EMB], w, dn, preferred_element_type=jnp.float32)
    hi = lax.dot_general(
        p[:, D_EMB : 2 * D_EMB], w, dn, preferred_element_type=jnp.float32
    )
    for g in range(_G):
        o_ref[2 * g * _CB2 : (2 * g + 1) * _CB2, :] = lo[g * _CB2 : (g + 1) * _CB2]
        o_ref[(2 * g + 1) * _CB2 : (2 * g + 2) * _CB2, :] = hi[g * _CB2 : (g + 1) * _CB2]


def _tc_proj(e2, w):
    n2 = e2.shape[0]
    return pl.pallas_call(
        _proj_body,
        grid=(n2 // (_G * _CB2),),
        in_specs=[
            pl.BlockSpec((_G * _CB2, 2 * D_EMB), lambda i: (i, 0)),
            pl.BlockSpec((D_MODEL, D_EMB), lambda i: (0, 0)),
        ],
        out_specs=pl.BlockSpec((2 * _G * _CB2, D_MODEL), lambda i: (i, 0)),
        out_shape=jax.ShapeDtypeStruct((2 * n2, D_MODEL), jnp.float32),
    )(e2, w)


def _proj_body_acc(e2_ref, w_ref, dummy_ref, o_ref):
    del dummy_ref
    _proj_body(e2_ref, w_ref, o_ref)


def _tc_proj_chunk(e2c, w, prev, blk_off, n_total):
    n2 = e2c.shape[0]
    in_specs = [
        pl.BlockSpec((_G * _CB2, 2 * D_EMB), lambda i: (i, 0)),
        pl.BlockSpec((D_MODEL, D_EMB), lambda i: (0, 0)),
    ]
    out_spec = pl.BlockSpec((2 * _G * _CB2, D_MODEL), lambda i: (i + blk_off, 0))
    out_shape = jax.ShapeDtypeStruct((n_total, D_MODEL), jnp.float32)
    if prev is None:
        return pl.pallas_call(
            _proj_body,
            grid=(n2 // (_G * _CB2),),
            in_specs=in_specs,
            out_specs=out_spec,
            out_shape=out_shape,
        )(e2c, w)
    return pl.pallas_call(
        _proj_body_acc,
        grid=(n2 // (_G * _CB2),),
        in_specs=in_specs + [pl.BlockSpec(memory_space=pl.ANY)],
        out_specs=out_spec,
        out_shape=out_shape,
        input_output_aliases={2: 0},
    )(e2c, w, prev)


_NCHUNK = 5


def kernel(input_ids, embed_table, proj_weight):
    b, t = input_ids.shape
    n = b * t
    ids2 = (input_ids.reshape(-1) * 2).astype(jnp.int32)
    table_pp = _tc_transpose(jnp.swapaxes(embed_table, 0, 1))
    table2 = table_pp.reshape(2 * embed_table.shape[0], D_EMB)
    n_c = n // _NCHUNK
    blocks_per_chunk = n_c // (2 * _G * _CB2)
    out = None
    for c in range(_NCHUNK):
        e2c = _sc_gather(
            lax.slice(ids2, (c * n_c,), ((c + 1) * n_c,)), table2
        )
        out = _tc_proj_chunk(e2c, proj_weight, out, c * blocks_per_chunk, n)
    return out.reshape(b, t, D_MODEL)


# speedup vs baseline: 1.0220x; 1.0220x over previous
import functools

import jax
import jax.numpy as jnp
from jax import lax
from jax.experimental import pallas as pl
from jax.experimental.pallas import tpu as pltpu
from jax.experimental.pallas import tpu_sc as plsc

D_EMB = 64
D_PAD = 128
D_MODEL = 1024

_NC = 2
_NS = 16
_NW = _NC * _NS

_CB2 = 512
_GB = 128


_VB = 4096


def _tr_body(t_ref, o_ref):
    x = t_ref[...]
    r = lax.broadcasted_iota(jnp.int32, (D_EMB, D_EMB), 0)
    c = lax.broadcasted_iota(jnp.int32, (D_EMB, D_EMB), 1)
    ident = (r == c).astype(jnp.float32)
    xt = lax.dot_general(
        x, ident, (((0,), (0,)), ((), ())), preferred_element_type=jnp.float32
    )
    o_ref[:, 0:D_EMB] = xt[0:_VB]
    o_ref[:, D_EMB : 2 * D_EMB] = xt[_VB : 2 * _VB]


def _tc_transpose(table_t):
    v = table_t.shape[1]
    nblk = pl.cdiv(v, 2 * _VB)
    return pl.pallas_call(
        _tr_body,
        grid=(nblk,),
        in_specs=[pl.BlockSpec((D_EMB, 2 * _VB), lambda i: (0, i))],
        out_specs=pl.BlockSpec((_VB, D_PAD), lambda i: (i, 0)),
        out_shape=jax.ShapeDtypeStruct((nblk * _VB, D_PAD), jnp.float32),
    )(table_t)


def _pack_ids(ids):
    slab = ids // (2 * _VB)
    u = ids % (2 * _VB)
    return 2 * _VB * slab + jnp.where(u < _VB, 2 * u, 2 * (u - _VB) + 1)


def _gather_body(ids_hbm, table_hbm, out_hbm, idx_v, rows_v, sem, n_per_w):
    wid = lax.axis_index("s") * _NC + lax.axis_index("c")
    base = wid * n_per_w
    pltpu.sync_copy(ids_hbm.at[pl.ds(base, n_per_w)], idx_v)

    def body(i, carry):
        tok = i * (2 * _CB2)
        for half in range(2):
            for j in range(_CB2 // _GB):
                pltpu.async_copy(
                    table_hbm.at[idx_v.at[pl.ds(tok + half * _CB2 + j * _GB, _GB)]],
                    rows_v.at[pl.ds((half * _CB2 + j * _GB), _GB)],
                    sem,
                )
        for half in range(2):
            for j in range(_CB2 // _GB):
                pltpu.make_async_copy(
                    table_hbm.at[idx_v.at[pl.ds(tok + half * _CB2 + j * _GB, _GB)]],
                    rows_v.at[pl.ds((half * _CB2 + j * _GB), _GB)],
                    sem,
                ).wait()
        prow = base // 2 + i * _CB2
        pltpu.sync_copy(
            rows_v.at[pl.ds(0, _CB2)],
            out_hbm.at[pl.ds(prow, _CB2), pl.ds(0, D_EMB)],
        )
        pltpu.sync_copy(
            rows_v.at[pl.ds(_CB2, _CB2)],
            out_hbm.at[pl.ds(prow, _CB2), pl.ds(D_EMB, D_EMB)],
        )
        return carry

    lax.fori_loop(0, n_per_w // (2 * _CB2), body, 0)


def _sc_gather(ids2, table2):
    n = ids2.shape[0]
    n_per_w = n // _NW
    mesh = plsc.VectorSubcoreMesh(core_axis_name="c", subcore_axis_name="s")
    k = pl.kernel(
        functools.partial(_gather_body, n_per_w=n_per_w),
        out_type=jax.ShapeDtypeStruct((n // 2, 2 * D_EMB), jnp.float32),
        mesh=mesh,
        scratch_types=[
            pltpu.VMEM((n_per_w,), jnp.int32),
            pltpu.VMEM((2 * _CB2, D_EMB), jnp.float32),
            pltpu.SemaphoreType.DMA,
        ],
        compiler_params=pltpu.CompilerParams(use_tc_tiling_on_sc=False),
    )
    return k(ids2, table2)


_G = 4


def _proj_body(e2_ref, w_ref, o_ref):
    p = e2_ref[...]
    w = w_ref[...]
    dn = (((1,), (1,)), ((), ()))
    lo = lax.dot_general(p[:, 0:D_EMB], w, dn, preferred_element_type=jnp.float32)
    hi = lax.dot_general(
        p[:, D_EMB : 2 * D_EMB], w, dn, preferred_element_type=jnp.float32
    )
    for g in range(_G):
        o_ref[2 * g * _CB2 : (2 * g + 1) * _CB2, :] = lo[g * _CB2 : (g + 1) * _CB2]
        o_ref[(2 * g + 1) * _CB2 : (2 * g + 2) * _CB2, :] = hi[g * _CB2 : (g + 1) * _CB2]


def _tc_proj(e2, w):
    n2 = e2.shape[0]
    return pl.pallas_call(
        _proj_body,
        grid=(n2 // (_G * _CB2),),
        in_specs=[
            pl.BlockSpec((_G * _CB2, 2 * D_EMB), lambda i: (i, 0)),
            pl.BlockSpec((D_MODEL, D_EMB), lambda i: (0, 0)),
        ],
        out_specs=pl.BlockSpec((2 * _G * _CB2, D_MODEL), lambda i: (i, 0)),
        out_shape=jax.ShapeDtypeStruct((2 * n2, D_MODEL), jnp.float32),
    )(e2, w)


def kernel(input_ids, embed_table, proj_weight):
    b, t = input_ids.shape
    ids2 = _pack_ids(input_ids.reshape(-1).astype(jnp.int32))
    table_pp = _tc_transpose(jnp.swapaxes(embed_table, 0, 1))
    table2 = table_pp.reshape(2 * table_pp.shape[0], D_EMB)
    e2 = _sc_gather(ids2, table2)
    out = _tc_proj(e2, proj_weight)
    return out.reshape(b, t, D_MODEL)


# speedup vs baseline: 1.0428x; 1.0204x over previous
import functools

import jax
import jax.numpy as jnp
from jax import lax
from jax.experimental import pallas as pl
from jax.experimental.pallas import tpu as pltpu
from jax.experimental.pallas import tpu_sc as plsc

D_EMB = 64
D_PAD = 128
D_MODEL = 1024

_NC = 2
_NS = 16
_NW = _NC * _NS

_CB2 = 512
_GB = 128


_VB = 8192


def _tr_body(t_ref, o_ref):
    x = t_ref[...]
    r = lax.broadcasted_iota(jnp.int32, (D_EMB, D_EMB), 0)
    c = lax.broadcasted_iota(jnp.int32, (D_EMB, D_EMB), 1)
    ident = (r == c).astype(jnp.float32)
    xt = lax.dot_general(
        x, ident, (((0,), (0,)), ((), ())), preferred_element_type=jnp.float32
    )
    o_ref[:, 0:D_EMB] = xt[0:_VB]
    o_ref[:, D_EMB : 2 * D_EMB] = xt[_VB : 2 * _VB]


def _tc_transpose(table_t):
    v = table_t.shape[1]
    nblk = pl.cdiv(v, 2 * _VB)
    return pl.pallas_call(
        _tr_body,
        grid=(nblk,),
        in_specs=[pl.BlockSpec((D_EMB, 2 * _VB), lambda i: (0, i))],
        out_specs=pl.BlockSpec((_VB, D_PAD), lambda i: (i, 0)),
        out_shape=jax.ShapeDtypeStruct((nblk * _VB, D_PAD), jnp.float32),
    )(table_t)


def _pack_ids(ids):
    slab = ids // (2 * _VB)
    u = ids % (2 * _VB)
    return 2 * _VB * slab + jnp.where(u < _VB, 2 * u, 2 * (u - _VB) + 1)


def _gather_body(ids_hbm, table_hbm, out_hbm, idx_v, rows_v, sem, n_per_w):
    wid = lax.axis_index("s") * _NC + lax.axis_index("c")
    base = wid * n_per_w
    pltpu.sync_copy(ids_hbm.at[pl.ds(base, n_per_w)], idx_v)

    def body(i, carry):
        tok = i * (2 * _CB2)
        for half in range(2):
            for j in range(_CB2 // _GB):
                pltpu.async_copy(
                    table_hbm.at[idx_v.at[pl.ds(tok + half * _CB2 + j * _GB, _GB)]],
                    rows_v.at[pl.ds((half * _CB2 + j * _GB), _GB)],
                    sem,
                )
        for half in range(2):
            for j in range(_CB2 // _GB):
                pltpu.make_async_copy(
                    table_hbm.at[idx_v.at[pl.ds(tok + half * _CB2 + j * _GB, _GB)]],
                    rows_v.at[pl.ds((half * _CB2 + j * _GB), _GB)],
                    sem,
                ).wait()
        prow = base // 2 + i * _CB2
        pltpu.sync_copy(
            rows_v.at[pl.ds(0, _CB2)],
            out_hbm.at[pl.ds(prow, _CB2), pl.ds(0, D_EMB)],
        )
        pltpu.sync_copy(
            rows_v.at[pl.ds(_CB2, _CB2)],
            out_hbm.at[pl.ds(prow, _CB2), pl.ds(D_EMB, D_EMB)],
        )
        return carry

    lax.fori_loop(0, n_per_w // (2 * _CB2), body, 0)


def _sc_gather(ids2, table2):
    n = ids2.shape[0]
    n_per_w = n // _NW
    mesh = plsc.VectorSubcoreMesh(core_axis_name="c", subcore_axis_name="s")
    k = pl.kernel(
        functools.partial(_gather_body, n_per_w=n_per_w),
        out_type=jax.ShapeDtypeStruct((n // 2, 2 * D_EMB), jnp.float32),
        mesh=mesh,
        scratch_types=[
            pltpu.VMEM((n_per_w,), jnp.int32),
            pltpu.VMEM((2 * _CB2, D_EMB), jnp.float32),
            pltpu.SemaphoreType.DMA,
        ],
        compiler_params=pltpu.CompilerParams(use_tc_tiling_on_sc=False),
    )
    return k(ids2, table2)


_G = 4


def _proj_body(e2_ref, w_ref, o_ref):
    p = e2_ref[...]
    w = w_ref[...]
    dn = (((1,), (1,)), ((), ()))
    lo = lax.dot_general(p[:, 0:D_EMB], w, dn, preferred_element_type=jnp.float32)
    hi = lax.dot_general(
        p[:, D_EMB : 2 * D_EMB], w, dn, preferred_element_type=jnp.float32
    )
    for g in range(_G):
        o_ref[2 * g * _CB2 : (2 * g + 1) * _CB2, :] = lo[g * _CB2 : (g + 1) * _CB2]
        o_ref[(2 * g + 1) * _CB2 : (2 * g + 2) * _CB2, :] = hi[g * _CB2 : (g + 1) * _CB2]


def _tc_proj(e2, w):
    n2 = e2.shape[0]
    return pl.pallas_call(
        _proj_body,
        grid=(n2 // (_G * _CB2),),
        in_specs=[
            pl.BlockSpec((_G * _CB2, 2 * D_EMB), lambda i: (i, 0)),
            pl.BlockSpec((D_MODEL, D_EMB), lambda i: (0, 0)),
        ],
        out_specs=pl.BlockSpec((2 * _G * _CB2, D_MODEL), lambda i: (i, 0)),
        out_shape=jax.ShapeDtypeStruct((2 * n2, D_MODEL), jnp.float32),
    )(e2, w)


def kernel(input_ids, embed_table, proj_weight):
    b, t = input_ids.shape
    ids2 = _pack_ids(input_ids.reshape(-1).astype(jnp.int32))
    table_pp = _tc_transpose(jnp.swapaxes(embed_table, 0, 1))
    table2 = table_pp.reshape(2 * table_pp.shape[0], D_EMB)
    e2 = _sc_gather(ids2, table2)
    out = _tc_proj(e2, proj_weight)
    return out.reshape(b, t, D_MODEL)


# speedup vs baseline: 1.0523x; 1.0091x over previous
import functools

import jax
import jax.numpy as jnp
from jax import lax
from jax.experimental import pallas as pl
from jax.experimental.pallas import tpu as pltpu
from jax.experimental.pallas import tpu_sc as plsc

D_EMB = 64
D_PAD = 128
D_MODEL = 1024

_NC = 2
_NS = 16
_NW = _NC * _NS

_CB2 = 512
_GB = 128


_VB = 16384


def _tr_body(t_ref, o_ref):
    x = t_ref[...]
    r = lax.broadcasted_iota(jnp.int32, (D_EMB, D_EMB), 0)
    c = lax.broadcasted_iota(jnp.int32, (D_EMB, D_EMB), 1)
    ident = (r == c).astype(jnp.float32)
    xt = lax.dot_general(
        x, ident, (((0,), (0,)), ((), ())), preferred_element_type=jnp.float32
    )
    o_ref[:, 0:D_EMB] = xt[0:_VB]
    o_ref[:, D_EMB : 2 * D_EMB] = xt[_VB : 2 * _VB]


def _tc_transpose(table_t):
    v = table_t.shape[1]
    nblk = pl.cdiv(v, 2 * _VB)
    return pl.pallas_call(
        _tr_body,
        grid=(nblk,),
        in_specs=[pl.BlockSpec((D_EMB, 2 * _VB), lambda i: (0, i))],
        out_specs=pl.BlockSpec((_VB, D_PAD), lambda i: (i, 0)),
        out_shape=jax.ShapeDtypeStruct((nblk * _VB, D_PAD), jnp.float32),
    )(table_t)


def _pack_ids(ids):
    slab = ids // (2 * _VB)
    u = ids % (2 * _VB)
    return 2 * _VB * slab + jnp.where(u < _VB, 2 * u, 2 * (u - _VB) + 1)


def _gather_body(ids_hbm, table_hbm, out_hbm, idx_v, rows_v, sem, n_per_w):
    wid = lax.axis_index("s") * _NC + lax.axis_index("c")
    base = wid * n_per_w
    pltpu.sync_copy(ids_hbm.at[pl.ds(base, n_per_w)], idx_v)

    def body(i, carry):
        tok = i * (2 * _CB2)
        for half in range(2):
            for j in range(_CB2 // _GB):
                pltpu.async_copy(
                    table_hbm.at[idx_v.at[pl.ds(tok + half * _CB2 + j * _GB, _GB)]],
                    rows_v.at[pl.ds((half * _CB2 + j * _GB), _GB)],
                    sem,
                )
        for half in range(2):
            for j in range(_CB2 // _GB):
                pltpu.make_async_copy(
                    table_hbm.at[idx_v.at[pl.ds(tok + half * _CB2 + j * _GB, _GB)]],
                    rows_v.at[pl.ds((half * _CB2 + j * _GB), _GB)],
                    sem,
                ).wait()
        prow = base // 2 + i * _CB2
        pltpu.sync_copy(
            rows_v.at[pl.ds(0, _CB2)],
            out_hbm.at[pl.ds(prow, _CB2), pl.ds(0, D_EMB)],
        )
        pltpu.sync_copy(
            rows_v.at[pl.ds(_CB2, _CB2)],
            out_hbm.at[pl.ds(prow, _CB2), pl.ds(D_EMB, D_EMB)],
        )
        return carry

    lax.fori_loop(0, n_per_w // (2 * _CB2), body, 0)


def _sc_gather(ids2, table2):
    n = ids2.shape[0]
    n_per_w = n // _NW
    mesh = plsc.VectorSubcoreMesh(core_axis_name="c", subcore_axis_name="s")
    k = pl.kernel(
        functools.partial(_gather_body, n_per_w=n_per_w),
        out_type=jax.ShapeDtypeStruct((n // 2, 2 * D_EMB), jnp.float32),
        mesh=mesh,
        scratch_types=[
            pltpu.VMEM((n_per_w,), jnp.int32),
            pltpu.VMEM((2 * _CB2, D_EMB), jnp.float32),
            pltpu.SemaphoreType.DMA,
        ],
        compiler_params=pltpu.CompilerParams(use_tc_tiling_on_sc=False),
    )
    return k(ids2, table2)


_G = 4


def _proj_body(e2_ref, w_ref, o_ref):
    p = e2_ref[...]
    w = w_ref[...]
    dn = (((1,), (1,)), ((), ()))
    lo = lax.dot_general(p[:, 0:D_EMB], w, dn, preferred_element_type=jnp.float32)
    hi = lax.dot_general(
        p[:, D_EMB : 2 * D_EMB], w, dn, preferred_element_type=jnp.float32
    )
    for g in range(_G):
        o_ref[2 * g * _CB2 : (2 * g + 1) * _CB2, :] = lo[g * _CB2 : (g + 1) * _CB2]
        o_ref[(2 * g + 1) * _CB2 : (2 * g + 2) * _CB2, :] = hi[g * _CB2 : (g + 1) * _CB2]


def _tc_proj(e2, w):
    n2 = e2.shape[0]
    return pl.pallas_call(
        _proj_body,
        grid=(n2 // (_G * _CB2),),
        in_specs=[
            pl.BlockSpec((_G * _CB2, 2 * D_EMB), lambda i: (i, 0)),
            pl.BlockSpec((D_MODEL, D_EMB), lambda i: (0, 0)),
        ],
        out_specs=pl.BlockSpec((2 * _G * _CB2, D_MODEL), lambda i: (i, 0)),
        out_shape=jax.ShapeDtypeStruct((2 * n2, D_MODEL), jnp.float32),
    )(e2, w)


def kernel(input_ids, embed_table, proj_weight):
    b, t = input_ids.shape
    ids2 = _pack_ids(input_ids.reshape(-1).astype(jnp.int32))
    table_pp = _tc_transpose(jnp.swapaxes(embed_table, 0, 1))
    table2 = table_pp.reshape(2 * table_pp.shape[0], D_EMB)
    e2 = _sc_gather(ids2, table2)
    out = _tc_proj(e2, proj_weight)
    return out.reshape(b, t, D_MODEL)
